# Initial kernel scaffold; baseline (speedup 1.0000x reference)
import jax
import jax.numpy as jnp
from jax.experimental import pallas as pl


def kernel(qk, attention_mask_, bucket_size):
    raise NotImplementedError("write your pallas kernel here")


# speedup vs baseline: 3.2488x; 3.2488x over previous
import functools

import jax
import jax.numpy as jnp
from jax import lax
from jax.experimental import pallas as pl

_K = 32
_LANES = 128


def _rnd16(x):
    b = lax.bitcast_convert_type(x, jnp.int32)
    neg = b < 0
    absb = jnp.bitwise_and(b, 0x7FFFFFFF)
    e = absb >> 23
    norm = ((absb + 0xFFF + jnp.bitwise_and(absb >> 13, 1)) >> 13) - 0x1C000
    ax = jnp.abs(x)
    subf = (ax * jnp.float32(16777216.0) + jnp.float32(8388608.0)
            ) - jnp.float32(8388608.0)
    sub = subf.astype(jnp.int32)
    cand = jnp.where(e < 113, sub, norm)
    cand = jnp.minimum(cand, 0x7C00)
    cand = jnp.where(absb > 0x7F800000, 0x7E00, cand)
    val_norm = lax.bitcast_convert_type((cand + 0x1C000) << 13, jnp.float32)
    val_sub = cand.astype(jnp.float32) * jnp.float32(2.0**-24)
    val = jnp.where(cand >= 0x400, val_norm, val_sub)
    val = jnp.where(cand >= 0x7C00, jnp.float32(jnp.inf), val)
    val = jnp.where(neg, -val, val)
    bits16 = jnp.where(neg, jnp.bitwise_or(cand, 0x8000), cand)
    return bits16, val


def _mask_body(a_ref, p_ref, qs_ref, mask_ref, out_ref):
    a_blk = a_ref[...]
    p_blk = p_ref[...]
    qs = qs_ref[0]
    msk = mask_ref[...]

    pres = lax.dot_general(
        a_blk, p_blk, (((1,), (1,)), ((), ())),
        preferred_element_type=jnp.float32)
    _, pres16 = _rnd16(pres)
    t = pres16 * qs
    r_bits, r_val = _rnd16(t)
    bad = jnp.bitwise_and(r_bits, 0x7FFF) > 0x7C00
    r_val = jnp.where(bad, jnp.float32(0.0), r_val)
    s = r_val + msk
    bits, _ = _rnd16(s)

    u = jnp.bitwise_and(bits, 0xFFFF)
    ordered = jnp.where(u < 0x8000, u + 0x8000, 0xFFFF - u)
    col = lax.broadcasted_iota(jnp.int32, u.shape, 1)
    key = jnp.bitwise_or(ordered << 12, 4095 - col)

    def body(_, carry):
        k, selected = carry
        m = jnp.max(k, axis=1, keepdims=True)
        sel = k == m
        return jnp.where(sel, -1, k), jnp.where(sel, 1, selected)

    _, selected = lax.fori_loop(
        0, _K, body, (key, jnp.zeros(u.shape, jnp.int32)))

    out_ref[...] = jnp.where(selected == 1,
                             jnp.float32(0.0), jnp.float32(-10000.0))


@functools.partial(jax.jit, static_argnums=(4, 5))
def _topk_mask(a_pad, p_pad, qs, mask_row, s, block_r):
    hh = a_pad.shape[0] // s
    nr = s // block_r
    grid = (hh, nr)
    return pl.pallas_call(
        _mask_body,
        grid=grid,
        in_specs=[
            pl.BlockSpec((block_r, _LANES), lambda h, r: (h * nr + r, 0)),
            pl.BlockSpec((s, _LANES), lambda h, r: (h, 0)),
            pl.BlockSpec((1, 1, s), lambda h, r: (h, 0, 0)),
            pl.BlockSpec((1, s), lambda h, r: (0, 0)),
        ],
        out_specs=pl.BlockSpec((block_r, s), lambda h, r: (h * nr + r, 0)),
        out_shape=jax.ShapeDtypeStruct((hh * s, s), jnp.float32),
    )(a_pad, p_pad, qs, mask_row)


def kernel(qk, attention_mask_, bucket_size):
    qk = lax.stop_gradient(qk).astype(jnp.float16)
    attention_mask = attention_mask_.astype(jnp.float16)
    batch_size, n_heads, seq_len, dim = qk.shape
    qk_norm = jnp.linalg.norm(qk, axis=-1, keepdims=True)
    phi = jnp.max(qk_norm)
    qk_const = jnp.sqrt(jnp.power(phi, 2) - jnp.power(qk_norm, 2))
    tmp_zero = jnp.zeros(qk_const.shape, dtype=jnp.float16)
    Q = jnp.concatenate((qk, tmp_zero), axis=-1)
    P = jnp.concatenate((qk, qk_const), axis=-1)
    _P_norm = jnp.linalg.norm(P, axis=-1, keepdims=True)
    _Q_norm = jnp.linalg.norm(Q, axis=-1, keepdims=True)
    _M = jnp.max(_P_norm)
    P = P / _P_norm * _M
    Q = Q / _Q_norm * _M
    a = jax.random.normal(jax.random.key(42),
                          (batch_size, n_heads, seq_len, dim + 1),
                          dtype=jnp.float32).astype(jnp.float16)
    Qs = jnp.sum(Q * a, axis=-1)

    hh = batch_size * n_heads
    pad = _LANES - (dim + 1)
    a_pad = jnp.pad(a.astype(jnp.float32).reshape(hh * seq_len, dim + 1),
                    ((0, 0), (0, pad)))
    p_pad = jnp.pad(P.astype(jnp.float32).reshape(hh * seq_len, dim + 1),
                    ((0, 0), (0, pad)))
    qs3 = Qs.reshape(hh, 1, seq_len).astype(jnp.float32)
    mask_row = attention_mask.reshape(1, seq_len).astype(jnp.float32)

    block_r = 256 if seq_len % 256 == 0 else seq_len
    out = _topk_mask(a_pad, p_pad, qs3, mask_row, seq_len, block_r)
    return lax.stop_gradient(
        out.reshape(batch_size, n_heads, seq_len, seq_len))


# speedup vs baseline: 4.1903x; 1.2898x over previous
import functools

import jax
import jax.numpy as jnp
from jax import lax
from jax.experimental import pallas as pl

_K = 32
_LANES = 128


def _rnd16(x):
    b = lax.bitcast_convert_type(x, jnp.int32)
    neg = b < 0
    absb = jnp.bitwise_and(b, 0x7FFFFFFF)
    e = absb >> 23
    norm = ((absb + 0xFFF + jnp.bitwise_and(absb >> 13, 1)) >> 13) - 0x1C000
    m = jnp.bitwise_or(jnp.bitwise_and(absb, 0x7FFFFF), 0x800000)
    sh = jnp.clip(126 - e, 0, 31)
    base = lax.shift_right_logical(m, sh)
    low = jnp.bitwise_and(m, lax.shift_left(jnp.int32(1), sh) - 1)
    half = lax.shift_left(jnp.int32(1), jnp.maximum(sh, 1) - 1)
    incr = jnp.logical_or(
        low > half,
        jnp.logical_and(low == half, jnp.bitwise_and(base, 1) == 1))
    sub = base + incr.astype(jnp.int32)
    cand = jnp.where(e < 113, sub, norm)
    cand = jnp.minimum(cand, 0x7C00)
    cand = jnp.where(absb > 0x7F800000, 0x7E00, cand)
    val_norm = lax.bitcast_convert_type((cand + 0x1C000) << 13, jnp.float32)
    val_sub = cand.astype(jnp.float32) * jnp.float32(2.0**-24)
    val = jnp.where(cand >= 0x400, val_norm, val_sub)
    val = jnp.where(cand >= 0x7C00, jnp.float32(jnp.inf), val)
    val = jnp.where(neg, -val, val)
    bits16 = jnp.where(neg, jnp.bitwise_or(cand, 0x8000), cand)
    return bits16, val


def _r16v(x):
    return _rnd16(x)[1]


def _mask_body(a_ref, p_ref, qs_ref, mask_ref, out_ref):
    a_blk = a_ref[...]
    p_blk = p_ref[...]
    qs = qs_ref[0]
    msk = mask_ref[...]

    pres = lax.dot_general(
        a_blk, p_blk, (((1,), (1,)), ((), ())),
        preferred_element_type=jnp.float32)
    _, pres16 = _rnd16(pres)
    t = pres16 * qs
    r_bits, r_val = _rnd16(t)
    bad = jnp.bitwise_and(r_bits, 0x7FFF) > 0x7C00
    r_val = jnp.where(bad, jnp.float32(0.0), r_val)
    s = r_val + msk
    bits, _ = _rnd16(s)

    u = jnp.bitwise_and(bits, 0xFFFF)
    ordered = jnp.where(u < 0x8000, u + 0x8000, 0xFFFF - u)
    col = lax.broadcasted_iota(jnp.int32, u.shape, 1)
    key = jnp.bitwise_or(ordered << 12, 4095 - col)

    ones_v = jnp.ones((u.shape[1], 1), jnp.float32)

    def body(i, p):
        c = jnp.bitwise_or(p, lax.shift_left(jnp.int32(1), 27 - i))
        ge = (key >= c).astype(jnp.float32)
        cnt = lax.dot_general(ge, ones_v, (((1,), (0,)), ((), ())),
                              preferred_element_type=jnp.float32)
        return jnp.where(cnt >= jnp.float32(_K), c, p)

    p = lax.fori_loop(0, 28, body,
                      jnp.zeros((u.shape[0], 1), jnp.int32))

    out_ref[...] = jnp.where(key >= p,
                             jnp.float32(0.0), jnp.float32(-10000.0))


@functools.partial(jax.jit, static_argnums=(4, 5))
def _topk_mask(a_pad, p_pad, qs, mask_row, s, block_r):
    hh = a_pad.shape[0] // s
    nr = s // block_r
    grid = (hh, nr)
    return pl.pallas_call(
        _mask_body,
        grid=grid,
        in_specs=[
            pl.BlockSpec((block_r, _LANES), lambda h, r: (h * nr + r, 0)),
            pl.BlockSpec((s, _LANES), lambda h, r: (h, 0)),
            pl.BlockSpec((1, 1, s), lambda h, r: (h, 0, 0)),
            pl.BlockSpec((1, s), lambda h, r: (0, 0)),
        ],
        out_specs=pl.BlockSpec((block_r, s), lambda h, r: (h * nr + r, 0)),
        out_shape=jax.ShapeDtypeStruct((hh * s, s), jnp.float32),
    )(a_pad, p_pad, qs, mask_row)


def kernel(qk, attention_mask_, bucket_size):
    qk = lax.stop_gradient(qk).astype(jnp.float16)
    attention_mask = attention_mask_.astype(jnp.float16)
    batch_size, n_heads, seq_len, dim = qk.shape
    qk_norm = jnp.linalg.norm(qk, axis=-1, keepdims=True)
    phi = jnp.max(qk_norm)
    qk_const = jnp.sqrt(jnp.power(phi, 2) - jnp.power(qk_norm, 2))
    tmp_zero = jnp.zeros(qk_const.shape, dtype=jnp.float16)
    Q = jnp.concatenate((qk, tmp_zero), axis=-1)
    P = jnp.concatenate((qk, qk_const), axis=-1)
    _P_norm = jnp.linalg.norm(P, axis=-1, keepdims=True)
    _Q_norm = jnp.linalg.norm(Q, axis=-1, keepdims=True)
    _M = jnp.max(_P_norm)
    P = P / _P_norm * _M
    Q = Q / _Q_norm * _M
    a_raw = jax.random.normal(jax.random.key(42),
                              (batch_size, n_heads, seq_len, dim + 1),
                              dtype=jnp.float32)
    a32 = _r16v(a_raw)
    q32 = _r16v(Q.astype(jnp.float32))
    p32 = _r16v(P.astype(jnp.float32))
    prod = _r16v(q32 * a32)
    qs32 = _r16v(jnp.sum(prod, axis=-1))

    hh = batch_size * n_heads
    pad = _LANES - (dim + 1)
    a_pad = jnp.pad(a32.reshape(hh * seq_len, dim + 1), ((0, 0), (0, pad)))
    p_pad = jnp.pad(p32.reshape(hh * seq_len, dim + 1), ((0, 0), (0, pad)))
    qs3 = qs32.reshape(hh, 1, seq_len)
    mask_row = _r16v(attention_mask_.astype(jnp.float32)).reshape(1, seq_len)

    block_r = 256 if seq_len % 256 == 0 else seq_len
    out = _topk_mask(a_pad, p_pad, qs3, mask_row, seq_len, block_r)
    return lax.stop_gradient(
        out.reshape(batch_size, n_heads, seq_len, seq_len))


# speedup vs baseline: 7.0322x; 1.6782x over previous
import functools

import jax
import jax.numpy as jnp
from jax import lax
from jax.experimental import pallas as pl

_K = 32
_LANES = 128


def _rnd16(x):
    b = lax.bitcast_convert_type(x, jnp.int32)
    neg = b < 0
    absb = jnp.bitwise_and(b, 0x7FFFFFFF)
    e = absb >> 23
    norm = ((absb + 0xFFF + jnp.bitwise_and(absb >> 13, 1)) >> 13) - 0x1C000
    m = jnp.bitwise_or(jnp.bitwise_and(absb, 0x7FFFFF), 0x800000)
    sh = jnp.clip(126 - e, 0, 31)
    base = lax.shift_right_logical(m, sh)
    low = jnp.bitwise_and(m, lax.shift_left(jnp.int32(1), sh) - 1)
    half = lax.shift_left(jnp.int32(1), jnp.maximum(sh, 1) - 1)
    incr = jnp.logical_or(
        low > half,
        jnp.logical_and(low == half, jnp.bitwise_and(base, 1) == 1))
    sub = base + incr.astype(jnp.int32)
    cand = jnp.where(e < 113, sub, norm)
    cand = jnp.minimum(cand, 0x7C00)
    cand = jnp.where(absb > 0x7F800000, 0x7E00, cand)
    val_norm = lax.bitcast_convert_type((cand + 0x1C000) << 13, jnp.float32)
    val_sub = cand.astype(jnp.float32) * jnp.float32(2.0**-24)
    val = jnp.where(cand >= 0x400, val_norm, val_sub)
    val = jnp.where(cand >= 0x7C00, jnp.float32(jnp.inf), val)
    val = jnp.where(neg, -val, val)
    bits16 = jnp.where(neg, jnp.bitwise_or(cand, 0x8000), cand)
    return bits16, val


def _r16v(x):
    return _rnd16(x)[1]


def _f16_round_fast(x):
    b = lax.bitcast_convert_type(x, jnp.int32)
    eb = jnp.bitwise_and(b, 0x7F800000)
    c = lax.bitcast_convert_type(eb, jnp.float32) * jnp.float32(12288.0)
    c = jnp.maximum(c, jnp.float32(0.75))
    c = jnp.where(b < 0, -c, c)
    r = (x + c) - c
    r = jnp.where(jnp.bitwise_and(b, 0x7FFFFFFF) >= 0x7F800000, x, r)
    return jnp.where(jnp.abs(r) > jnp.float32(65504.0),
                     jnp.where(b < 0, jnp.float32(-jnp.inf),
                               jnp.float32(jnp.inf)), r)


def _mask_body(a_ref, p_ref, qs_ref, mask_ref, out_ref):
    a_blk = a_ref[...]
    p_blk = p_ref[...]
    qs = qs_ref[0]
    msk = mask_ref[...]

    pres = lax.dot_general(
        a_blk, p_blk, (((1,), (1,)), ((), ())),
        preferred_element_type=jnp.float32)
    pres16 = _f16_round_fast(pres)
    t = pres16 * qs
    r_val = _f16_round_fast(t)
    r_val = jnp.where(t != t, jnp.float32(0.0), r_val)
    s = r_val + msk
    sv = _f16_round_fast(s)

    sb = lax.bitcast_convert_type(sv, jnp.int32)
    mag = jnp.clip((jnp.bitwise_and(sb, 0x7FFFFFFF) >> 13) - 0x1C000,
                   0, 0x7C00)
    ordered = jnp.where(sb < 0, 0x7FFF - mag, 0x8000 + mag)
    col = lax.broadcasted_iota(jnp.int32, ordered.shape, 1)
    key = jnp.bitwise_or(ordered << 12, 4095 - col)

    ones_v = jnp.ones((ordered.shape[1], 1), jnp.float32)

    def body(i, p):
        c = jnp.bitwise_or(p, lax.shift_left(jnp.int32(1), 27 - i))
        ge = (key >= c).astype(jnp.float32)
        cnt = lax.dot_general(ge, ones_v, (((1,), (0,)), ((), ())),
                              preferred_element_type=jnp.float32)
        return jnp.where(cnt >= jnp.float32(_K), c, p)

    p = lax.fori_loop(0, 28, body,
                      jnp.zeros((ordered.shape[0], 1), jnp.int32))

    out_ref[...] = jnp.where(key >= p,
                             jnp.float32(0.0), jnp.float32(-10000.0))


@functools.partial(jax.jit, static_argnums=(4, 5))
def _topk_mask(a_pad, p_pad, qs, mask_row, s, block_r):
    hh = a_pad.shape[0] // s
    nr = s // block_r
    grid = (hh, nr)
    return pl.pallas_call(
        _mask_body,
        grid=grid,
        in_specs=[
            pl.BlockSpec((block_r, _LANES), lambda h, r: (h * nr + r, 0)),
            pl.BlockSpec((s, _LANES), lambda h, r: (h, 0)),
            pl.BlockSpec((1, 1, s), lambda h, r: (h, 0, 0)),
            pl.BlockSpec((1, s), lambda h, r: (0, 0)),
        ],
        out_specs=pl.BlockSpec((block_r, s), lambda h, r: (h * nr + r, 0)),
        out_shape=jax.ShapeDtypeStruct((hh * s, s), jnp.float32),
    )(a_pad, p_pad, qs, mask_row)


def kernel(qk, attention_mask_, bucket_size):
    qk = lax.stop_gradient(qk).astype(jnp.float16)
    attention_mask = attention_mask_.astype(jnp.float16)
    batch_size, n_heads, seq_len, dim = qk.shape
    qk_norm = jnp.linalg.norm(qk, axis=-1, keepdims=True)
    phi = jnp.max(qk_norm)
    qk_const = jnp.sqrt(jnp.power(phi, 2) - jnp.power(qk_norm, 2))
    tmp_zero = jnp.zeros(qk_const.shape, dtype=jnp.float16)
    Q = jnp.concatenate((qk, tmp_zero), axis=-1)
    P = jnp.concatenate((qk, qk_const), axis=-1)
    _P_norm = jnp.linalg.norm(P, axis=-1, keepdims=True)
    _Q_norm = jnp.linalg.norm(Q, axis=-1, keepdims=True)
    _M = jnp.max(_P_norm)
    P = P / _P_norm * _M
    Q = Q / _Q_norm * _M
    a_raw = jax.random.normal(jax.random.key(42),
                              (batch_size, n_heads, seq_len, dim + 1),
                              dtype=jnp.float32)
    a32 = _r16v(a_raw)
    q32 = _r16v(Q.astype(jnp.float32))
    p32 = _r16v(P.astype(jnp.float32))
    prod = _r16v(q32 * a32)
    qs32 = _r16v(jnp.sum(prod, axis=-1))

    hh = batch_size * n_heads
    pad = _LANES - (dim + 1)
    a_pad = jnp.pad(a32.reshape(hh * seq_len, dim + 1), ((0, 0), (0, pad)))
    p_pad = jnp.pad(p32.reshape(hh * seq_len, dim + 1), ((0, 0), (0, pad)))
    qs3 = qs32.reshape(hh, 1, seq_len)
    mask_row = _r16v(attention_mask_.astype(jnp.float32)).reshape(1, seq_len)

    block_r = 256 if seq_len % 256 == 0 else seq_len
    out = _topk_mask(a_pad, p_pad, qs3, mask_row, seq_len, block_r)
    return lax.stop_gradient(
        out.reshape(batch_size, n_heads, seq_len, seq_len))


# speedup vs baseline: 8.2400x; 1.1718x over previous
import functools

import jax
import jax.numpy as jnp
from jax import lax
from jax.experimental import pallas as pl

_K = 32
_LANES = 128


def _rnd16(x):
    b = lax.bitcast_convert_type(x, jnp.int32)
    neg = b < 0
    absb = jnp.bitwise_and(b, 0x7FFFFFFF)
    e = absb >> 23
    norm = ((absb + 0xFFF + jnp.bitwise_and(absb >> 13, 1)) >> 13) - 0x1C000
    m = jnp.bitwise_or(jnp.bitwise_and(absb, 0x7FFFFF), 0x800000)
    sh = jnp.clip(126 - e, 0, 31)
    base = lax.shift_right_logical(m, sh)
    low = jnp.bitwise_and(m, lax.shift_left(jnp.int32(1), sh) - 1)
    half = lax.shift_left(jnp.int32(1), jnp.maximum(sh, 1) - 1)
    incr = jnp.logical_or(
        low > half,
        jnp.logical_and(low == half, jnp.bitwise_and(base, 1) == 1))
    sub = base + incr.astype(jnp.int32)
    cand = jnp.where(e < 113, sub, norm)
    cand = jnp.minimum(cand, 0x7C00)
    cand = jnp.where(absb > 0x7F800000, 0x7E00, cand)
    val_norm = lax.bitcast_convert_type((cand + 0x1C000) << 13, jnp.float32)
    val_sub = cand.astype(jnp.float32) * jnp.float32(2.0**-24)
    val = jnp.where(cand >= 0x400, val_norm, val_sub)
    val = jnp.where(cand >= 0x7C00, jnp.float32(jnp.inf), val)
    val = jnp.where(neg, -val, val)
    bits16 = jnp.where(neg, jnp.bitwise_or(cand, 0x8000), cand)
    return bits16, val


def _r16v(x):
    return _rnd16(x)[1]


def _f16_round_fast(x):
    b = lax.bitcast_convert_type(x, jnp.int32)
    eb = jnp.bitwise_and(b, 0x7F800000)
    c = lax.bitcast_convert_type(eb, jnp.float32) * jnp.float32(12288.0)
    c = jnp.maximum(c, jnp.float32(0.75))
    c = jnp.where(b < 0, -c, c)
    r = (x + c) - c
    r = jnp.where(jnp.bitwise_and(b, 0x7FFFFFFF) >= 0x7F800000, x, r)
    return jnp.where(jnp.abs(r) > jnp.float32(65504.0),
                     jnp.where(b < 0, jnp.float32(-jnp.inf),
                               jnp.float32(jnp.inf)), r)


def _mask_body(a_ref, p_ref, qs_ref, mask_ref, out_ref):
    a_blk = a_ref[...]
    p_blk = p_ref[...]
    qs = qs_ref[0]
    msk = mask_ref[...]

    pres = lax.dot_general(
        a_blk, p_blk, (((1,), (1,)), ((), ())),
        preferred_element_type=jnp.float32)
    pres16 = _f16_round_fast(pres)
    t = pres16 * qs
    r_val = _f16_round_fast(t)
    r_val = jnp.where(t != t, jnp.float32(0.0), r_val)
    s = r_val + msk
    sv = _f16_round_fast(s)

    sb = lax.bitcast_convert_type(sv, jnp.int32)
    mag = jnp.clip((jnp.bitwise_and(sb, 0x7FFFFFFF) >> 13) - 0x1C000,
                   0, 0x7C00)
    ordered = jnp.where(sb < 0, 0x7FFF - mag, 0x8000 + mag)
    col = lax.broadcasted_iota(jnp.int32, ordered.shape, 1)
    key = jnp.bitwise_or(ordered << 12, 4095 - col)

    ones_v = jnp.ones((ordered.shape[1], 1), jnp.float32)

    def body(i, p):
        c = jnp.bitwise_or(p, lax.shift_left(jnp.int32(1), 27 - i))
        ge = (key >= c).astype(jnp.float32)
        cnt = lax.dot_general(ge, ones_v, (((1,), (0,)), ((), ())),
                              preferred_element_type=jnp.float32)
        return jnp.where(cnt >= jnp.float32(_K), c, p)

    p = lax.fori_loop(0, 28, body,
                      jnp.zeros((ordered.shape[0], 1), jnp.int32))

    out_ref[...] = jnp.where(key >= p,
                             jnp.float32(0.0), jnp.float32(-10000.0))


@functools.partial(jax.jit, static_argnums=(4, 5))
def _topk_mask(a_pad, p_pad, qs, mask_row, s, block_r):
    hh = a_pad.shape[0] // s
    nr = s // block_r
    grid = (hh, nr)
    return pl.pallas_call(
        _mask_body,
        grid=grid,
        in_specs=[
            pl.BlockSpec((block_r, _LANES), lambda h, r: (h * nr + r, 0)),
            pl.BlockSpec((s, _LANES), lambda h, r: (h, 0)),
            pl.BlockSpec((1, 1, s), lambda h, r: (h, 0, 0)),
            pl.BlockSpec((1, s), lambda h, r: (0, 0)),
        ],
        out_specs=pl.BlockSpec((block_r, s), lambda h, r: (h * nr + r, 0)),
        out_shape=jax.ShapeDtypeStruct((hh * s, s), jnp.float32),
    )(a_pad, p_pad, qs, mask_row)


def kernel(qk, attention_mask_, bucket_size):
    qk = lax.stop_gradient(qk).astype(jnp.float16)
    attention_mask = attention_mask_.astype(jnp.float16)
    batch_size, n_heads, seq_len, dim = qk.shape
    qk_norm = jnp.linalg.norm(qk, axis=-1, keepdims=True)
    phi = jnp.max(qk_norm)
    qk_const = jnp.sqrt(jnp.power(phi, 2) - jnp.power(qk_norm, 2))
    tmp_zero = jnp.zeros(qk_const.shape, dtype=jnp.float16)
    Q = jnp.concatenate((qk, tmp_zero), axis=-1)
    P = jnp.concatenate((qk, qk_const), axis=-1)
    _P_norm = jnp.linalg.norm(P, axis=-1, keepdims=True)
    _Q_norm = jnp.linalg.norm(Q, axis=-1, keepdims=True)
    _M = jnp.max(_P_norm)
    P = P / _P_norm * _M
    Q = Q / _Q_norm * _M
    a_raw = jax.random.normal(jax.random.key(42),
                              (batch_size, n_heads, seq_len, dim + 1),
                              dtype=jnp.float32)
    a32 = _r16v(a_raw)
    q32 = _r16v(Q.astype(jnp.float32))
    p32 = _r16v(P.astype(jnp.float32))
    prod = _r16v(q32 * a32)
    qs32 = _r16v(jnp.sum(prod, axis=-1))

    hh = batch_size * n_heads
    pad = _LANES - (dim + 1)
    a_pad = jnp.pad(a32.reshape(hh * seq_len, dim + 1), ((0, 0), (0, pad)))
    p_pad = jnp.pad(p32.reshape(hh * seq_len, dim + 1), ((0, 0), (0, pad)))
    qs3 = qs32.reshape(hh, 1, seq_len)
    mask_row = _r16v(attention_mask_.astype(jnp.float32)).reshape(1, seq_len)

    block_r = 512 if seq_len % 512 == 0 else seq_len
    out = _topk_mask(a_pad, p_pad, qs3, mask_row, seq_len, block_r)
    return lax.stop_gradient(
        out.reshape(batch_size, n_heads, seq_len, seq_len))


# speedup vs baseline: 8.9930x; 1.0914x over previous
import functools

import jax
import jax.numpy as jnp
from jax import lax
from jax.experimental import pallas as pl

_K = 32
_LANES = 128


def _rnd16(x):
    b = lax.bitcast_convert_type(x, jnp.int32)
    neg = b < 0
    absb = jnp.bitwise_and(b, 0x7FFFFFFF)
    e = absb >> 23
    norm = ((absb + 0xFFF + jnp.bitwise_and(absb >> 13, 1)) >> 13) - 0x1C000
    m = jnp.bitwise_or(jnp.bitwise_and(absb, 0x7FFFFF), 0x800000)
    sh = jnp.clip(126 - e, 0, 31)
    base = lax.shift_right_logical(m, sh)
    low = jnp.bitwise_and(m, lax.shift_left(jnp.int32(1), sh) - 1)
    half = lax.shift_left(jnp.int32(1), jnp.maximum(sh, 1) - 1)
    incr = jnp.logical_or(
        low > half,
        jnp.logical_and(low == half, jnp.bitwise_and(base, 1) == 1))
    sub = base + incr.astype(jnp.int32)
    cand = jnp.where(e < 113, sub, norm)
    cand = jnp.minimum(cand, 0x7C00)
    cand = jnp.where(absb > 0x7F800000, 0x7E00, cand)
    val_norm = lax.bitcast_convert_type((cand + 0x1C000) << 13, jnp.float32)
    val_sub = cand.astype(jnp.float32) * jnp.float32(2.0**-24)
    val = jnp.where(cand >= 0x400, val_norm, val_sub)
    val = jnp.where(cand >= 0x7C00, jnp.float32(jnp.inf), val)
    val = jnp.where(neg, -val, val)
    bits16 = jnp.where(neg, jnp.bitwise_or(cand, 0x8000), cand)
    return bits16, val


def _r16v(x):
    return _rnd16(x)[1]


def _f16_round_fast(x):
    b = lax.bitcast_convert_type(x, jnp.int32)
    eb = jnp.bitwise_and(b, 0x7F800000)
    c = lax.bitcast_convert_type(eb, jnp.float32) * jnp.float32(12288.0)
    c = jnp.maximum(c, jnp.float32(0.75))
    c = jnp.where(b < 0, -c, c)
    r = (x + c) - c
    r = jnp.where(jnp.bitwise_and(b, 0x7FFFFFFF) >= 0x7F800000, x, r)
    return jnp.where(jnp.abs(r) > jnp.float32(65504.0),
                     jnp.where(b < 0, jnp.float32(-jnp.inf),
                               jnp.float32(jnp.inf)), r)


def _mask_body(a_ref, p_ref, qs_ref, mask_ref, out_ref):
    a_blk = a_ref[...]
    p_blk = p_ref[...]
    qs = qs_ref[0]
    msk = mask_ref[...]

    pres = lax.dot_general(
        a_blk, p_blk, (((1,), (1,)), ((), ())),
        preferred_element_type=jnp.float32)
    pres16 = _f16_round_fast(pres)
    t = pres16 * qs
    r_val = _f16_round_fast(t)
    r_val = jnp.where(t != t, jnp.float32(0.0), r_val)
    s = r_val + msk
    sv = _f16_round_fast(s)

    sb = lax.bitcast_convert_type(sv, jnp.int32)
    mag = jnp.clip((jnp.bitwise_and(sb, 0x7FFFFFFF) >> 13) - 0x1C000,
                   0, 0x7C00)
    ordered = jnp.where(sb < 0, 0x7FFF - mag, 0x8000 + mag)
    col = lax.broadcasted_iota(jnp.int32, ordered.shape, 1)
    key = jnp.bitwise_or(ordered << 12, 4095 - col)

    ones_v = jnp.ones((ordered.shape[1], 1), jnp.float32)

    def body(i, p):
        c = jnp.bitwise_or(p, lax.shift_left(jnp.int32(1), 27 - i))
        ge = (key >= c).astype(jnp.float32)
        cnt = lax.dot_general(ge, ones_v, (((1,), (0,)), ((), ())),
                              preferred_element_type=jnp.float32)
        return jnp.where(cnt >= jnp.float32(_K), c, p)

    p = lax.fori_loop(0, 28, body,
                      jnp.zeros((ordered.shape[0], 1), jnp.int32))

    out_ref[...] = jnp.where(key >= p,
                             jnp.float32(0.0), jnp.float32(-10000.0))


@functools.partial(jax.jit, static_argnums=(4, 5))
def _topk_mask(a_pad, p_pad, qs, mask_row, s, block_r):
    hh = a_pad.shape[0] // s
    nr = s // block_r
    grid = (hh, nr)
    return pl.pallas_call(
        _mask_body,
        grid=grid,
        in_specs=[
            pl.BlockSpec((block_r, _LANES), lambda h, r: (h * nr + r, 0)),
            pl.BlockSpec((s, _LANES), lambda h, r: (h, 0)),
            pl.BlockSpec((1, 1, s), lambda h, r: (h, 0, 0)),
            pl.BlockSpec((1, s), lambda h, r: (0, 0)),
        ],
        out_specs=pl.BlockSpec((block_r, s), lambda h, r: (h * nr + r, 0)),
        out_shape=jax.ShapeDtypeStruct((hh * s, s), jnp.float32),
    )(a_pad, p_pad, qs, mask_row)


def kernel(qk, attention_mask_, bucket_size):
    qk = lax.stop_gradient(qk).astype(jnp.float16)
    attention_mask = attention_mask_.astype(jnp.float16)
    batch_size, n_heads, seq_len, dim = qk.shape
    qk_norm = jnp.linalg.norm(qk, axis=-1, keepdims=True)
    phi = jnp.max(qk_norm)
    qk_const = jnp.sqrt(jnp.power(phi, 2) - jnp.power(qk_norm, 2))
    tmp_zero = jnp.zeros(qk_const.shape, dtype=jnp.float16)
    Q = jnp.concatenate((qk, tmp_zero), axis=-1)
    P = jnp.concatenate((qk, qk_const), axis=-1)
    _P_norm = jnp.linalg.norm(P, axis=-1, keepdims=True)
    _Q_norm = jnp.linalg.norm(Q, axis=-1, keepdims=True)
    _M = jnp.max(_P_norm)
    P = P / _P_norm * _M
    Q = Q / _Q_norm * _M
    a_raw = jax.random.normal(jax.random.key(42),
                              (batch_size, n_heads, seq_len, dim + 1),
                              dtype=jnp.float32)
    a32 = _r16v(a_raw)
    q32 = _r16v(Q.astype(jnp.float32))
    p32 = _r16v(P.astype(jnp.float32))
    prod = _r16v(q32 * a32)
    qs32 = _r16v(jnp.sum(prod, axis=-1))

    hh = batch_size * n_heads
    pad = _LANES - (dim + 1)
    a_pad = jnp.pad(a32.reshape(hh * seq_len, dim + 1), ((0, 0), (0, pad)))
    p_pad = jnp.pad(p32.reshape(hh * seq_len, dim + 1), ((0, 0), (0, pad)))
    qs3 = qs32.reshape(hh, 1, seq_len)
    mask_row = _r16v(attention_mask_.astype(jnp.float32)).reshape(1, seq_len)

    block_r = 1024 if seq_len % 1024 == 0 else seq_len
    out = _topk_mask(a_pad, p_pad, qs3, mask_row, seq_len, block_r)
    return lax.stop_gradient(
        out.reshape(batch_size, n_heads, seq_len, seq_len))


# speedup vs baseline: 9.0050x; 1.0013x over previous
import functools

import jax
import jax.numpy as jnp
from jax import lax
from jax.experimental import pallas as pl

_K = 32
_LANES = 128


def _rnd16(x):
    b = lax.bitcast_convert_type(x, jnp.int32)
    neg = b < 0
    absb = jnp.bitwise_and(b, 0x7FFFFFFF)
    e = absb >> 23
    norm = ((absb + 0xFFF + jnp.bitwise_and(absb >> 13, 1)) >> 13) - 0x1C000
    m = jnp.bitwise_or(jnp.bitwise_and(absb, 0x7FFFFF), 0x800000)
    sh = jnp.clip(126 - e, 0, 31)
    base = lax.shift_right_logical(m, sh)
    low = jnp.bitwise_and(m, lax.shift_left(jnp.int32(1), sh) - 1)
    half = lax.shift_left(jnp.int32(1), jnp.maximum(sh, 1) - 1)
    incr = jnp.logical_or(
        low > half,
        jnp.logical_and(low == half, jnp.bitwise_and(base, 1) == 1))
    sub = base + incr.astype(jnp.int32)
    cand = jnp.where(e < 113, sub, norm)
    cand = jnp.minimum(cand, 0x7C00)
    cand = jnp.where(absb > 0x7F800000, 0x7E00, cand)
    val_norm = lax.bitcast_convert_type((cand + 0x1C000) << 13, jnp.float32)
    val_sub = cand.astype(jnp.float32) * jnp.float32(2.0**-24)
    val = jnp.where(cand >= 0x400, val_norm, val_sub)
    val = jnp.where(cand >= 0x7C00, jnp.float32(jnp.inf), val)
    val = jnp.where(neg, -val, val)
    bits16 = jnp.where(neg, jnp.bitwise_or(cand, 0x8000), cand)
    return bits16, val


def _r16v(x):
    return _rnd16(x)[1]


def _f16_round_fast(x):
    b = lax.bitcast_convert_type(x, jnp.int32)
    eb = jnp.bitwise_and(b, 0x7F800000)
    c = lax.bitcast_convert_type(eb, jnp.float32) * jnp.float32(12288.0)
    c = jnp.maximum(c, jnp.float32(0.75))
    c = jnp.where(b < 0, -c, c)
    r = (x + c) - c
    r = jnp.where(jnp.bitwise_and(b, 0x7FFFFFFF) >= 0x7F800000, x, r)
    return jnp.where(jnp.abs(r) > jnp.float32(65504.0),
                     jnp.where(b < 0, jnp.float32(-jnp.inf),
                               jnp.float32(jnp.inf)), r)


def _mask_body(a_ref, p_ref, qs_ref, mask_ref, out_ref):
    a_blk = a_ref[...]
    p_blk = p_ref[...]
    qs = qs_ref[0]
    msk = mask_ref[...]

    pres = lax.dot_general(
        a_blk, p_blk, (((1,), (1,)), ((), ())),
        preferred_element_type=jnp.float32)
    pres16 = _f16_round_fast(pres)
    t = pres16 * qs
    r_val = _f16_round_fast(t)
    r_val = jnp.where(t != t, jnp.float32(0.0), r_val)
    s = r_val + msk
    sv = _f16_round_fast(s)

    sb = lax.bitcast_convert_type(sv, jnp.int32)
    mag = jnp.clip((jnp.bitwise_and(sb, 0x7FFFFFFF) >> 13) - 0x1C000,
                   0, 0x7C00)
    ordered = jnp.where(sb < 0, 0x7FFF - mag, 0x8000 + mag)
    col = lax.broadcasted_iota(jnp.int32, ordered.shape, 1)
    key = jnp.bitwise_or(ordered << 12, 4095 - col)

    ones_v = jnp.ones((ordered.shape[1], 1), jnp.bfloat16)

    def body(i, p):
        c = jnp.bitwise_or(p, lax.shift_left(jnp.int32(1), 27 - i))
        ge = (key >= c).astype(jnp.bfloat16)
        cnt = lax.dot_general(ge, ones_v, (((1,), (0,)), ((), ())),
                              preferred_element_type=jnp.float32)
        return jnp.where(cnt >= jnp.float32(_K), c, p)

    p = lax.fori_loop(0, 28, body,
                      jnp.zeros((ordered.shape[0], 1), jnp.int32))

    out_ref[...] = jnp.where(key >= p,
                             jnp.float32(0.0), jnp.float32(-10000.0))


@functools.partial(jax.jit, static_argnums=(4, 5))
def _topk_mask(a_pad, p_pad, qs, mask_row, s, block_r):
    hh = a_pad.shape[0] // s
    nr = s // block_r
    grid = (hh, nr)
    return pl.pallas_call(
        _mask_body,
        grid=grid,
        in_specs=[
            pl.BlockSpec((block_r, _LANES), lambda h, r: (h * nr + r, 0)),
            pl.BlockSpec((s, _LANES), lambda h, r: (h, 0)),
            pl.BlockSpec((1, 1, s), lambda h, r: (h, 0, 0)),
            pl.BlockSpec((1, s), lambda h, r: (0, 0)),
        ],
        out_specs=pl.BlockSpec((block_r, s), lambda h, r: (h * nr + r, 0)),
        out_shape=jax.ShapeDtypeStruct((hh * s, s), jnp.float32),
    )(a_pad, p_pad, qs, mask_row)


def kernel(qk, attention_mask_, bucket_size):
    qk = lax.stop_gradient(qk).astype(jnp.float16)
    attention_mask = attention_mask_.astype(jnp.float16)
    batch_size, n_heads, seq_len, dim = qk.shape
    qk_norm = jnp.linalg.norm(qk, axis=-1, keepdims=True)
    phi = jnp.max(qk_norm)
    qk_const = jnp.sqrt(jnp.power(phi, 2) - jnp.power(qk_norm, 2))
    tmp_zero = jnp.zeros(qk_const.shape, dtype=jnp.float16)
    Q = jnp.concatenate((qk, tmp_zero), axis=-1)
    P = jnp.concatenate((qk, qk_const), axis=-1)
    _P_norm = jnp.linalg.norm(P, axis=-1, keepdims=True)
    _Q_norm = jnp.linalg.norm(Q, axis=-1, keepdims=True)
    _M = jnp.max(_P_norm)
    P = P / _P_norm * _M
    Q = Q / _Q_norm * _M
    a_raw = jax.random.normal(jax.random.key(42),
                              (batch_size, n_heads, seq_len, dim + 1),
                              dtype=jnp.float32)
    a32 = _r16v(a_raw)
    q32 = _r16v(Q.astype(jnp.float32))
    p32 = _r16v(P.astype(jnp.float32))
    prod = _r16v(q32 * a32)
    qs32 = _r16v(jnp.sum(prod, axis=-1))

    hh = batch_size * n_heads
    pad = _LANES - (dim + 1)
    a_pad = jnp.pad(a32.reshape(hh * seq_len, dim + 1), ((0, 0), (0, pad)))
    p_pad = jnp.pad(p32.reshape(hh * seq_len, dim + 1), ((0, 0), (0, pad)))
    qs3 = qs32.reshape(hh, 1, seq_len)
    mask_row = _r16v(attention_mask_.astype(jnp.float32)).reshape(1, seq_len)

    block_r = 1024 if seq_len % 1024 == 0 else seq_len
    out = _topk_mask(a_pad, p_pad, qs3, mask_row, seq_len, block_r)
    return lax.stop_gradient(
        out.reshape(batch_size, n_heads, seq_len, seq_len))
